# baseline (speedup 1.0000x reference)
import functools

import jax
import jax.numpy as jnp
from jax.experimental import pallas as pl
from jax.experimental.pallas import tpu as pltpu


def _fusion_kernel(hid_ref, dy_ref, w1t_ref, b1_ref, w2_ref, out_ref):
    hid = hid_ref[...]
    dy = dy_ref[...]
    w1t = w1t_ref[...]
    b1 = b1_ref[...]
    w2 = w2_ref[...]

    d = w1t.shape[0]
    chunk = 128
    acc_h = None
    acc_d = None
    for c in range(0, d, chunk):
        w1c = w1t[:, c:c + chunk]
        b1c = b1[:, c:c + chunk]
        w2c = w2[:, c:c + chunk]
        h_hc = jnp.tanh(jnp.dot(hid, w1c, preferred_element_type=jnp.float32) + b1c)
        h_dc = jnp.tanh(jnp.dot(dy, w1c, preferred_element_type=jnp.float32) + b1c)
        ph = h_hc * w2c
        pd = h_dc * w2c
        acc_h = ph if acc_h is None else acc_h + ph
        acc_d = pd if acc_d is None else acc_d + pd
    s_h = jnp.sum(acc_h, axis=1, keepdims=True)
    s_d = jnp.sum(acc_d, axis=1, keepdims=True)
    a = jax.nn.sigmoid(s_h - s_d)
    out_ref[...] = dy + a * (hid - dy)


@jax.jit
def kernel(hidden, dy_emb, W1, b1, W2, b2):
    n, d = hidden.shape
    tile = 2000
    assert n % tile == 0
    grid = (n // tile,)

    w1t = W1.T
    b1r = b1.reshape(1, d)
    w2r = W2.reshape(1, d)

    return pl.pallas_call(
        _fusion_kernel,
        grid=grid,
        in_specs=[
            pl.BlockSpec((tile, d), lambda i: (i, 0)),
            pl.BlockSpec((tile, d), lambda i: (i, 0)),
            pl.BlockSpec((d, d), lambda i: (0, 0)),
            pl.BlockSpec((1, d), lambda i: (0, 0)),
            pl.BlockSpec((1, d), lambda i: (0, 0)),
        ],
        out_specs=pl.BlockSpec((tile, d), lambda i: (i, 0)),
        out_shape=jax.ShapeDtypeStruct((n, d), jnp.float32),
        compiler_params=pltpu.CompilerParams(
            dimension_semantics=("parallel",),
        ),
    )(hidden, dy_emb, w1t, b1r, w2r)


# speedup vs baseline: 1.1129x; 1.1129x over previous
import functools

import jax
import jax.numpy as jnp
from jax.experimental import pallas as pl
from jax.experimental.pallas import tpu as pltpu


def _fusion_kernel(hid_ref, dy_ref, w1t_ref, b1_ref, w2_ref, out_ref):
    hid = hid_ref[...]
    dy = dy_ref[...]
    w1t = w1t_ref[...]
    b1 = b1_ref[...]
    w2 = w2_ref[...]

    h_h = jnp.tanh(jnp.dot(hid, w1t, preferred_element_type=jnp.float32) + b1)
    h_d = jnp.tanh(jnp.dot(dy, w1t, preferred_element_type=jnp.float32) + b1)
    s_h = jnp.dot(h_h, w2, preferred_element_type=jnp.float32)[:, 0:1]
    s_d = jnp.dot(h_d, w2, preferred_element_type=jnp.float32)[:, 0:1]
    a = jax.nn.sigmoid(s_h - s_d)
    out_ref[...] = dy + a * (hid - dy)


@jax.jit
def kernel(hidden, dy_emb, W1, b1, W2, b2):
    n, d = hidden.shape
    tile = 2000
    assert n % tile == 0
    grid = (n // tile,)

    w1t = W1.T
    b1r = b1.reshape(1, d)
    w2c = jnp.pad(W2.reshape(d, 1), ((0, 0), (0, 127)))

    return pl.pallas_call(
        _fusion_kernel,
        grid=grid,
        in_specs=[
            pl.BlockSpec((tile, d), lambda i: (i, 0)),
            pl.BlockSpec((tile, d), lambda i: (i, 0)),
            pl.BlockSpec((d, d), lambda i: (0, 0)),
            pl.BlockSpec((1, d), lambda i: (0, 0)),
            pl.BlockSpec((d, 128), lambda i: (0, 0)),
        ],
        out_specs=pl.BlockSpec((tile, d), lambda i: (i, 0)),
        out_shape=jax.ShapeDtypeStruct((n, d), jnp.float32),
        compiler_params=pltpu.CompilerParams(
            dimension_semantics=("parallel",),
        ),
    )(hidden, dy_emb, w1t, b1r, w2c)


# speedup vs baseline: 1.2312x; 1.1063x over previous
import functools

import jax
import jax.numpy as jnp
from jax.experimental import pallas as pl
from jax.experimental.pallas import tpu as pltpu


def _fusion_kernel(hid_ref, dy_ref, w1t_ref, b1_ref, w2_ref, out_ref):
    hid = hid_ref[...]
    dy = dy_ref[...]
    w1t = w1t_ref[...]
    b1 = b1_ref[...]
    w2 = w2_ref[...]

    w1t_b = w1t.astype(jnp.bfloat16)
    h_h = jnp.tanh(
        jnp.dot(hid.astype(jnp.bfloat16), w1t_b,
                preferred_element_type=jnp.float32) + b1)
    h_d = jnp.tanh(
        jnp.dot(dy.astype(jnp.bfloat16), w1t_b,
                preferred_element_type=jnp.float32) + b1)
    s_h = jnp.sum(h_h * w2, axis=1, keepdims=True)
    s_d = jnp.sum(h_d * w2, axis=1, keepdims=True)
    a = jax.nn.sigmoid(s_h - s_d)
    out_ref[...] = dy + a * (hid - dy)


@jax.jit
def kernel(hidden, dy_emb, W1, b1, W2, b2):
    n, d = hidden.shape
    tile = 2000
    assert n % tile == 0
    grid = (n // tile,)

    w1t = W1.T
    b1r = b1.reshape(1, d)
    w2r = W2.reshape(1, d)

    return pl.pallas_call(
        _fusion_kernel,
        grid=grid,
        in_specs=[
            pl.BlockSpec((tile, d), lambda i: (i, 0)),
            pl.BlockSpec((tile, d), lambda i: (i, 0)),
            pl.BlockSpec((d, d), lambda i: (0, 0)),
            pl.BlockSpec((1, d), lambda i: (0, 0)),
            pl.BlockSpec((1, d), lambda i: (0, 0)),
        ],
        out_specs=pl.BlockSpec((tile, d), lambda i: (i, 0)),
        out_shape=jax.ShapeDtypeStruct((n, d), jnp.float32),
        compiler_params=pltpu.CompilerParams(
            dimension_semantics=("parallel",),
        ),
    )(hidden, dy_emb, w1t, b1r, w2r)


# speedup vs baseline: 1.2343x; 1.0025x over previous
import functools

import jax
import jax.numpy as jnp
from jax.experimental import pallas as pl
from jax.experimental.pallas import tpu as pltpu


def _fusion_kernel(hid_ref, dy_ref, w1t_ref, b1_ref, w2_ref, out_ref,
                   hid_b_ref, dy_b_ref, w1t_b_ref):
    hid = hid_ref[...]
    dy = dy_ref[...]
    b1 = b1_ref[...]
    w2 = w2_ref[...]

    hid_b_ref[...] = hid.astype(jnp.bfloat16)
    dy_b_ref[...] = dy.astype(jnp.bfloat16)
    w1t_b_ref[...] = w1t_ref[...].astype(jnp.bfloat16)
    h_h = jnp.tanh(
        jnp.dot(hid_b_ref[...], w1t_b_ref[...],
                preferred_element_type=jnp.float32) + b1)
    h_d = jnp.tanh(
        jnp.dot(dy_b_ref[...], w1t_b_ref[...],
                preferred_element_type=jnp.float32) + b1)
    s_h = jnp.sum(h_h * w2, axis=1, keepdims=True)
    s_d = jnp.sum(h_d * w2, axis=1, keepdims=True)
    a = jax.nn.sigmoid(s_h - s_d)
    out_ref[...] = dy + a * (hid - dy)


@jax.jit
def kernel(hidden, dy_emb, W1, b1, W2, b2):
    n, d = hidden.shape
    tile = 2000
    assert n % tile == 0
    grid = (n // tile,)

    w1t = W1.T
    b1r = b1.reshape(1, d)
    w2r = W2.reshape(1, d)

    return pl.pallas_call(
        _fusion_kernel,
        grid=grid,
        in_specs=[
            pl.BlockSpec((tile, d), lambda i: (i, 0)),
            pl.BlockSpec((tile, d), lambda i: (i, 0)),
            pl.BlockSpec((d, d), lambda i: (0, 0)),
            pl.BlockSpec((1, d), lambda i: (0, 0)),
            pl.BlockSpec((1, d), lambda i: (0, 0)),
        ],
        out_specs=pl.BlockSpec((tile, d), lambda i: (i, 0)),
        out_shape=jax.ShapeDtypeStruct((n, d), jnp.float32),
        scratch_shapes=[
            pltpu.VMEM((tile, d), jnp.bfloat16),
            pltpu.VMEM((tile, d), jnp.bfloat16),
            pltpu.VMEM((d, d), jnp.bfloat16),
        ],
        compiler_params=pltpu.CompilerParams(
            dimension_semantics=("parallel",),
        ),
    )(hidden, dy_emb, w1t, b1r, w2r)
